# baseline (speedup 1.0000x reference)
import functools

import jax
import jax.numpy as jnp
from jax import lax
from jax.experimental import pallas as pl
from jax.experimental.pallas import tpu as pltpu
from jax.experimental.pallas import tpu_sc as plsc

_K = 16
_BIGF = 1e30
_IBIG = 0x3FFFFFFF


def _bdot(a, b):
    return lax.dot_general(
        a.astype(jnp.bfloat16), b.astype(jnp.bfloat16),
        (((1,), (0,)), ((), ())),
        preferred_element_type=jnp.float32)


def _topk_body(xr_ref, xf_ref, idx_ref, cv_ref, nv_ref, ni_ref, flag_ref,
               *, n, r, c, nc):
    xr = xr_ref[...]
    xrb = xr.astype(jnp.bfloat16)
    sqr = jnp.sum(xr * xr, axis=1, keepdims=True)
    col_iota = lax.broadcasted_iota(jnp.int32, (r, c), 1)
    slot_i = lax.broadcasted_iota(jnp.int32, (r, _K), 1)

    def step(cb, carry):
        topv, topi = carry
        xc = xf_ref[pl.ds(cb * c, c), :]
        sqc = jnp.sum(xc * xc, axis=1)
        dot = lax.dot_general(xrb, xc.astype(jnp.bfloat16),
                              (((1,), (1,)), ((), ())),
                              preferred_element_type=jnp.float32)
        gcol = col_iota + cb * c
        dist = (sqr + sqc[None, :]) - 2.0 * dot
        dist = jnp.where(gcol < n, dist, _BIGF)
        dm = jnp.where(dist < topv[:, _K - 1:_K], dist, _BIGF)
        cv_ref[...] = dm
        nv_ref[...] = jnp.full((r, _K), _BIGF, jnp.float32)
        ni_ref[...] = jnp.full((r, _K), _IBIG, jnp.int32)
        flag_ref[0] = jnp.where(jnp.min(dm) < _BIGF, 1, 0).astype(jnp.int32)

        for s in range(_K):
            @pl.when(flag_ref[0] == 1)
            def _(s=s):
                cvv = cv_ref[...]
                m = jnp.min(cvv, axis=1, keepdims=True)
                j = jnp.min(jnp.where(cvv == m, gcol, _IBIG), axis=1,
                            keepdims=True)
                cv_ref[...] = jnp.where(gcol == j, _BIGF, cvv)
                put = slot_i == s
                nv_ref[...] = jnp.where(put, m, nv_ref[...])
                ni_ref[...] = jnp.where(put, j, ni_ref[...])
                flag_ref[0] = jnp.where(jnp.min(m) < _BIGF, 1, 0).astype(
                    jnp.int32)

        nv = nv_ref[...]
        ni = ni_ref[...]
        cnt_a = jnp.sum((nv[:, None, :] < topv[:, :, None]).astype(jnp.int32),
                        axis=2)
        cnt_b = jnp.sum((topv[:, None, :] <= nv[:, :, None]).astype(jnp.int32),
                        axis=2)
        rank_a = slot_i + cnt_a
        rank_b = slot_i + cnt_b
        eq_a = rank_a[:, :, None] == slot_i[:, None, :]
        eq_b = rank_b[:, :, None] == slot_i[:, None, :]
        mv = (jnp.sum(jnp.where(eq_a, topv[:, :, None], 0.0), axis=1)
              + jnp.sum(jnp.where(eq_b, nv[:, :, None], 0.0), axis=1))
        mi = (jnp.sum(jnp.where(eq_a, topi[:, :, None], 0), axis=1)
              + jnp.sum(jnp.where(eq_b, ni[:, :, None], 0), axis=1))
        return mv, mi

    topv0 = jnp.full((r, _K), _BIGF, jnp.float32)
    topi0 = jnp.full((r, _K), _IBIG, jnp.int32)
    _, topi = lax.fori_loop(0, nc, step, (topv0, topi0))
    idx_ref[...] = topi


def _topk(xpad, n, r=256, c=512):
    npad, d = xpad.shape
    nc = npad // c
    body = functools.partial(_topk_body, n=n, r=r, c=c, nc=nc)
    return pl.pallas_call(
        body,
        grid=(npad // r,),
        in_specs=[
            pl.BlockSpec((r, d), lambda i: (i, 0)),
            pl.BlockSpec((npad, d), lambda i: (0, 0)),
        ],
        out_specs=pl.BlockSpec((r, _K), lambda i: (i, 0)),
        out_shape=jax.ShapeDtypeStruct((npad, _K), jnp.int32),
        scratch_shapes=[
            pltpu.VMEM((r, c), jnp.float32),
            pltpu.VMEM((r, _K), jnp.float32),
            pltpu.VMEM((r, _K), jnp.int32),
            pltpu.SMEM((1,), jnp.int32),
        ],
    )(xpad, xpad)


def _pick_chunk(bw, dw):
    best = 8
    for cand in range(8, bw + 1, 8):
        if bw % cand == 0 and cand * (dw + 1) * 4 <= 516000:
            best = cand
    return best


def _gather_rows(table, idx):
    v, dw = table.shape
    b = idx.shape[0]
    info = plsc.get_sparse_core_info()
    nw = info.num_cores * info.num_subcores
    bw = b // nw
    chunk = _pick_chunk(bw, dw)
    nch = bw // chunk
    mesh = plsc.VectorSubcoreMesh(core_axis_name="c", subcore_axis_name="s")

    @functools.partial(
        pl.kernel, mesh=mesh,
        out_type=jax.ShapeDtypeStruct((b, dw), jnp.float32),
        compiler_params=pltpu.CompilerParams(use_tc_tiling_on_sc=False),
        scratch_types=[
            pltpu.VMEM((chunk,), jnp.int32),
            pltpu.VMEM((chunk, dw), jnp.float32),
            pltpu.SemaphoreType.DMA,
        ],
    )
    def gk(table_hbm, idx_hbm, out_hbm, idx_v, rows_v, sem):
        wid = lax.axis_index("s") * info.num_cores + lax.axis_index("c")
        base = wid * bw
        for ch in range(nch):
            off = base + ch * chunk
            pltpu.sync_copy(idx_hbm.at[pl.ds(off, chunk)], idx_v)
            pltpu.async_copy(table_hbm.at[idx_v], rows_v, sem).wait()
            pltpu.sync_copy(rows_v, out_hbm.at[pl.ds(off, chunk)])

    return gk(table, idx)


def _msg_max_body(x_ref, xg_ref, w1t_ref, w1b_ref, b1_ref, w2_ref, b2_ref,
                  o_ref, *, pad_to):
    xi = x_ref[...]
    w1b = w1b_ref[...].astype(jnp.bfloat16)
    w2 = w2_ref[...].astype(jnp.bfloat16)
    a = _bdot(xi, w1t_ref[...]) + b1_ref[...]
    acc = None
    for k in range(_K):
        diff = xg_ref[:, k, :] - xi
        z = lax.dot_general(diff.astype(jnp.bfloat16), w1b,
                            (((1,), (0,)), ((), ())),
                            preferred_element_type=jnp.float32) + a
        z = jax.nn.relu(z)
        z2 = lax.dot_general(z.astype(jnp.bfloat16), w2,
                             (((1,), (0,)), ((), ())),
                             preferred_element_type=jnp.float32)
        acc = z2 if acc is None else jnp.maximum(acc, z2)
    out = jax.nn.relu(acc + b2_ref[...])
    if pad_to is not None:
        rb = out.shape[0]
        out = jnp.concatenate(
            [out, jnp.zeros((rb, pad_to - out.shape[1]), jnp.float32)], axis=1)
    o_ref[...] = out


def _block_div(n, cap):
    best = 8
    for cand in range(8, cap + 1, 8):
        if n % cand == 0:
            best = cand
    return best


def _msg_max(xpad, xg, w1t, w1b, b1, w2, b2, pad_to=None):
    npad, d = xpad.shape
    n = xg.shape[0]
    h = w1t.shape[1]
    out_w = w2.shape[1] if pad_to is None else pad_to
    rb = _block_div(n, 512)
    body = functools.partial(_msg_max_body, pad_to=pad_to)
    return pl.pallas_call(
        body,
        grid=(n // rb,),
        in_specs=[
            pl.BlockSpec((rb, d), lambda i: (i, 0)),
            pl.BlockSpec((rb, _K, d), lambda i: (i, 0, 0)),
            pl.BlockSpec((d, h), lambda i: (0, 0)),
            pl.BlockSpec((d, h), lambda i: (0, 0)),
            pl.BlockSpec((1, h), lambda i: (0, 0)),
            pl.BlockSpec((h, w2.shape[1]), lambda i: (0, 0)),
            pl.BlockSpec((1, w2.shape[1]), lambda i: (0, 0)),
        ],
        out_specs=pl.BlockSpec((rb, out_w), lambda i: (i, 0)),
        out_shape=jax.ShapeDtypeStruct((npad, out_w), jnp.float32),
    )(xpad, xg, w1t, w1b, b1, w2, b2)


def _edge_mlp_body(src_ref, dst_ref, ea_ref, wsrc_ref, wdst_ref, wea_ref,
                   b1_ref, w2_ref, b2_ref, o_ref):
    z = (_bdot(src_ref[...], wsrc_ref[...])
         + _bdot(dst_ref[...], wdst_ref[...])
         + _bdot(ea_ref[...], wea_ref[...])
         + b1_ref[...])
    z = jax.nn.relu(z)
    zb = z.astype(jnp.bfloat16).astype(jnp.float32)
    wb = w2_ref[...].astype(jnp.bfloat16).astype(jnp.float32)
    o_ref[...] = jnp.sum(zb * wb, axis=1, keepdims=True) + b2_ref[...]


def _edge_mlp(g, ea6, wsrc, wdst, wea, b1, w2row, b2, e):
    eb = _block_div(e, 4096)
    nblk = e // eb
    return pl.pallas_call(
        _edge_mlp_body,
        grid=(nblk,),
        in_specs=[
            pl.BlockSpec((eb, 16), lambda i: (i, 0)),
            pl.BlockSpec((eb, 16), lambda i, nblk=nblk: (i + nblk, 0)),
            pl.BlockSpec((eb, 6), lambda i: (i, 0)),
            pl.BlockSpec((16, 16), lambda i: (0, 0)),
            pl.BlockSpec((16, 16), lambda i: (0, 0)),
            pl.BlockSpec((6, 16), lambda i: (0, 0)),
            pl.BlockSpec((1, 16), lambda i: (0, 0)),
            pl.BlockSpec((1, 16), lambda i: (0, 0)),
            pl.BlockSpec((1, 1), lambda i: (0, 0)),
        ],
        out_specs=pl.BlockSpec((eb, 1), lambda i: (i, 0)),
        out_shape=jax.ShapeDtypeStruct((e, 1), jnp.float32),
    )(g, g, ea6, wsrc, wdst, wea, b1, w2row, b2)


def _edge_conv(xpad, w1, b1, w2, b2, n, pad_to=None):
    npad, d = xpad.shape
    idx = _topk(xpad, n)
    idxf = idx[:n].reshape(-1)
    xg = _gather_rows(xpad, idxf).reshape(n, _K, d)
    return _msg_max(xpad, xg, w1[:d], w1[d:], b1.reshape(1, -1),
                    w2, b2.reshape(1, -1), pad_to=pad_to)


def kernel(x, edge_index, edge_attr, year, quarter,
           W1a, b1a, W2a, b2a,
           W1b, b1b, W2b, b2b,
           W1c, b1c, W2c, b2c,
           Wf1, bf1, Wf2, bf2):
    n, d = x.shape
    e = edge_index.shape[1]
    npad = ((n + 511) // 512) * 512

    xpad = jnp.pad(x, ((0, npad - n), (0, 0)))
    h1 = _edge_conv(xpad, W1a, b1a, W2a, b2a, n)
    h2 = _edge_conv(h1, W1b, b1b, W2b, b2b, n)
    h3 = _edge_conv(h2, W1c, b1c, W2c, b2c, n, pad_to=16)

    g = _gather_rows(h3, edge_index.reshape(-1))
    ea6 = jnp.concatenate([edge_attr, year, quarter], axis=1)
    wsrc = jnp.pad(Wf1[:8], ((0, 8), (0, 0)))
    wdst = jnp.pad(Wf1[8:16], ((0, 8), (0, 0)))
    wea = Wf1[16:22]
    out = _edge_mlp(g, ea6, wsrc, wdst, wea, bf1.reshape(1, -1),
                    Wf2.reshape(1, -1), bf2.reshape(1, 1), e)
    return out[:, 0]


# speedup vs baseline: 2.9251x; 2.9251x over previous
import functools

import jax
import jax.numpy as jnp
from jax import lax
from jax.experimental import pallas as pl
from jax.experimental.pallas import tpu as pltpu
from jax.experimental.pallas import tpu_sc as plsc

_K = 16
_BIGF = 1e30
_IBIG = 0x3FFFFFFF


def _bdot(a, b):
    return lax.dot_general(
        a.astype(jnp.bfloat16), b.astype(jnp.bfloat16),
        (((1,), (0,)), ((), ())),
        preferred_element_type=jnp.float32)


_G = 16


def _dist_groups_body(xr_ref, xf_ref, dist_ref, chunk_ref, gm_ref,
                      *, n, r, c, nc, ngrp):
    cb = pl.program_id(1)
    xr = xr_ref[...]
    xrb = xr.astype(jnp.bfloat16)
    sqr = jnp.sum(xr * xr, axis=1, keepdims=True)
    col_iota = lax.broadcasted_iota(jnp.int32, (r, c), 1)

    xc = xf_ref[...]
    sqc = jnp.sum(xc * xc, axis=1)
    dot = lax.dot_general(xrb, xc.astype(jnp.bfloat16),
                          (((1,), (1,)), ((), ())),
                          preferred_element_type=jnp.float32)
    gcol = col_iota + cb * c
    dist = (sqr + sqc[None, :]) - 2.0 * dot
    dist = jnp.where(gcol < n, dist, _BIGF)
    dist_ref[...] = dist
    gpb = c // _G
    gm_ref[:, pl.ds(cb * gpb, gpb)] = jnp.min(
        dist.reshape(r, gpb, _G), axis=2)

    @pl.when(cb == nc - 1)
    def _():
        gmv = gm_ref[...]
        gi = lax.broadcasted_iota(jnp.int32, (r, ngrp), 1)
        gs = []
        for _s in range(_K):
            m = jnp.min(gmv, axis=1, keepdims=True)
            g = jnp.min(jnp.where(gmv == m, gi, _IBIG), axis=1, keepdims=True)
            gs.append(g)
            gmv = jnp.where(gi == g, _BIGF, gmv)
        gsel = jnp.concatenate(gs, axis=1)
        crow = (pl.program_id(0) * r
                + lax.broadcasted_iota(jnp.int32, (r, _K), 0))
        chunk_ref[...] = crow * ngrp + gsel


def _topk_final_body(cand_ref, chunk_ref, idx_ref, *, rb, ngrp):
    cand = cand_ref[...]
    w = _K * _G
    lane = lax.broadcasted_iota(jnp.int32, (rb, w), 1)
    grp = chunk_ref[...] % ngrp
    gcol = (jnp.broadcast_to(grp[:, :, None], (rb, _K, _G)).reshape(rb, w) * _G
            + lane % _G)
    cv = cand
    js = []
    for _s in range(_K):
        m = jnp.min(cv, axis=1, keepdims=True)
        j = jnp.min(jnp.where(cv == m, gcol, _IBIG), axis=1, keepdims=True)
        js.append(j)
        cv = jnp.where(gcol == j, _BIGF, cv)
    idx_ref[...] = jnp.concatenate(js, axis=1)


def _topk(xpad, n, r=256, c=2048):
    npad, d = xpad.shape
    c = min(c, npad)
    nc = npad // c
    ngrp = npad // _G
    body = functools.partial(_dist_groups_body, n=n, r=r, c=c, nc=nc,
                             ngrp=ngrp)
    dist, chunkidx = pl.pallas_call(
        body,
        grid=(npad // r, nc),
        in_specs=[
            pl.BlockSpec((r, d), lambda i, cb: (i, 0)),
            pl.BlockSpec((c, d), lambda i, cb: (cb, 0)),
        ],
        out_specs=[
            pl.BlockSpec((r, c), lambda i, cb: (i, cb)),
            pl.BlockSpec((r, _K), lambda i, cb: (i, 0)),
        ],
        out_shape=[
            jax.ShapeDtypeStruct((npad, npad), jnp.float32),
            jax.ShapeDtypeStruct((npad, _K), jnp.int32),
        ],
        scratch_shapes=[pltpu.VMEM((r, ngrp), jnp.float32)],
    )(xpad, xpad)

    cidx = chunkidx[:n].reshape(-1)
    cand = _gather_rows(dist.reshape(npad * ngrp, _G), cidx)
    cand = cand.reshape(n, _K * _G)

    rb = _block_div(n, 512)
    fbody = functools.partial(_topk_final_body, rb=rb, ngrp=ngrp)
    return pl.pallas_call(
        fbody,
        grid=(n // rb,),
        in_specs=[
            pl.BlockSpec((rb, _K * _G), lambda i: (i, 0)),
            pl.BlockSpec((rb, _K), lambda i: (i, 0)),
        ],
        out_specs=pl.BlockSpec((rb, _K), lambda i: (i, 0)),
        out_shape=jax.ShapeDtypeStruct((n, _K), jnp.int32),
    )(cand, chunkidx[:n])


def _pick_chunk(bw, dw):
    best = 8
    for cand in range(8, bw + 1, 8):
        if bw % cand == 0 and cand * (dw + 1) * 4 <= 516000:
            best = cand
    return best


def _gather_rows(table, idx):
    v, dw = table.shape
    b = idx.shape[0]
    info = plsc.get_sparse_core_info()
    nw = info.num_cores * info.num_subcores
    bw = b // nw
    chunk = _pick_chunk(bw, dw)
    nch = bw // chunk
    mesh = plsc.VectorSubcoreMesh(core_axis_name="c", subcore_axis_name="s")

    @functools.partial(
        pl.kernel, mesh=mesh,
        out_type=jax.ShapeDtypeStruct((b, dw), jnp.float32),
        compiler_params=pltpu.CompilerParams(use_tc_tiling_on_sc=False),
        scratch_types=[
            pltpu.VMEM((chunk,), jnp.int32),
            pltpu.VMEM((chunk, dw), jnp.float32),
            pltpu.SemaphoreType.DMA,
        ],
    )
    def gk(table_hbm, idx_hbm, out_hbm, idx_v, rows_v, sem):
        wid = lax.axis_index("s") * info.num_cores + lax.axis_index("c")
        base = wid * bw
        for ch in range(nch):
            off = base + ch * chunk
            pltpu.sync_copy(idx_hbm.at[pl.ds(off, chunk)], idx_v)
            pltpu.async_copy(table_hbm.at[idx_v], rows_v, sem).wait()
            pltpu.sync_copy(rows_v, out_hbm.at[pl.ds(off, chunk)])

    return gk(table, idx)


def _msg_max_body(x_ref, xg_ref, w1t_ref, w1b_ref, b1_ref, w2_ref, b2_ref,
                  o_ref, *, pad_to):
    xi = x_ref[...]
    w1b = w1b_ref[...].astype(jnp.bfloat16)
    w2 = w2_ref[...].astype(jnp.bfloat16)
    a = _bdot(xi, w1t_ref[...]) + b1_ref[...]
    acc = None
    for k in range(_K):
        diff = xg_ref[:, k, :] - xi
        z = lax.dot_general(diff.astype(jnp.bfloat16), w1b,
                            (((1,), (0,)), ((), ())),
                            preferred_element_type=jnp.float32) + a
        z = jax.nn.relu(z)
        z2 = lax.dot_general(z.astype(jnp.bfloat16), w2,
                             (((1,), (0,)), ((), ())),
                             preferred_element_type=jnp.float32)
        acc = z2 if acc is None else jnp.maximum(acc, z2)
    out = jax.nn.relu(acc + b2_ref[...])
    if pad_to is not None:
        rb = out.shape[0]
        out = jnp.concatenate(
            [out, jnp.zeros((rb, pad_to - out.shape[1]), jnp.float32)], axis=1)
    o_ref[...] = out


def _block_div(n, cap):
    best = 8
    for cand in range(8, cap + 1, 8):
        if n % cand == 0:
            best = cand
    return best


def _msg_max(xpad, xg, w1t, w1b, b1, w2, b2, pad_to=None):
    npad, d = xpad.shape
    n = xg.shape[0]
    h = w1t.shape[1]
    out_w = w2.shape[1] if pad_to is None else pad_to
    rb = _block_div(n, 512)
    body = functools.partial(_msg_max_body, pad_to=pad_to)
    return pl.pallas_call(
        body,
        grid=(n // rb,),
        in_specs=[
            pl.BlockSpec((rb, d), lambda i: (i, 0)),
            pl.BlockSpec((rb, _K, d), lambda i: (i, 0, 0)),
            pl.BlockSpec((d, h), lambda i: (0, 0)),
            pl.BlockSpec((d, h), lambda i: (0, 0)),
            pl.BlockSpec((1, h), lambda i: (0, 0)),
            pl.BlockSpec((h, w2.shape[1]), lambda i: (0, 0)),
            pl.BlockSpec((1, w2.shape[1]), lambda i: (0, 0)),
        ],
        out_specs=pl.BlockSpec((rb, out_w), lambda i: (i, 0)),
        out_shape=jax.ShapeDtypeStruct((npad, out_w), jnp.float32),
    )(xpad, xg, w1t, w1b, b1, w2, b2)


def _edge_mlp_body(src_ref, dst_ref, ea_ref, wsrc_ref, wdst_ref, wea_ref,
                   b1_ref, w2_ref, b2_ref, o_ref):
    z = (_bdot(src_ref[...], wsrc_ref[...])
         + _bdot(dst_ref[...], wdst_ref[...])
         + _bdot(ea_ref[...], wea_ref[...])
         + b1_ref[...])
    z = jax.nn.relu(z)
    zb = z.astype(jnp.bfloat16).astype(jnp.float32)
    wb = w2_ref[...].astype(jnp.bfloat16).astype(jnp.float32)
    o_ref[...] = jnp.sum(zb * wb, axis=1, keepdims=True) + b2_ref[...]


def _edge_mlp(g, ea6, wsrc, wdst, wea, b1, w2row, b2, e):
    eb = _block_div(e, 4096)
    nblk = e // eb
    return pl.pallas_call(
        _edge_mlp_body,
        grid=(nblk,),
        in_specs=[
            pl.BlockSpec((eb, 16), lambda i: (i, 0)),
            pl.BlockSpec((eb, 16), lambda i, nblk=nblk: (i + nblk, 0)),
            pl.BlockSpec((eb, 6), lambda i: (i, 0)),
            pl.BlockSpec((16, 16), lambda i: (0, 0)),
            pl.BlockSpec((16, 16), lambda i: (0, 0)),
            pl.BlockSpec((6, 16), lambda i: (0, 0)),
            pl.BlockSpec((1, 16), lambda i: (0, 0)),
            pl.BlockSpec((1, 16), lambda i: (0, 0)),
            pl.BlockSpec((1, 1), lambda i: (0, 0)),
        ],
        out_specs=pl.BlockSpec((eb, 1), lambda i: (i, 0)),
        out_shape=jax.ShapeDtypeStruct((e, 1), jnp.float32),
    )(g, g, ea6, wsrc, wdst, wea, b1, w2row, b2)


def _edge_conv(xpad, w1, b1, w2, b2, n, pad_to=None):
    npad, d = xpad.shape
    idx = _topk(xpad, n)
    idxf = idx[:n].reshape(-1)
    xg = _gather_rows(xpad, idxf).reshape(n, _K, d)
    return _msg_max(xpad, xg, w1[:d], w1[d:], b1.reshape(1, -1),
                    w2, b2.reshape(1, -1), pad_to=pad_to)


def kernel(x, edge_index, edge_attr, year, quarter,
           W1a, b1a, W2a, b2a,
           W1b, b1b, W2b, b2b,
           W1c, b1c, W2c, b2c,
           Wf1, bf1, Wf2, bf2):
    n, d = x.shape
    e = edge_index.shape[1]
    npad = ((n + 511) // 512) * 512

    xpad = jnp.pad(x, ((0, npad - n), (0, 0)))
    h1 = _edge_conv(xpad, W1a, b1a, W2a, b2a, n)
    h2 = _edge_conv(h1, W1b, b1b, W2b, b2b, n)
    h3 = _edge_conv(h2, W1c, b1c, W2c, b2c, n, pad_to=16)

    g = _gather_rows(h3, edge_index.reshape(-1))
    ea6 = jnp.concatenate([edge_attr, year, quarter], axis=1)
    wsrc = jnp.pad(Wf1[:8], ((0, 8), (0, 0)))
    wdst = jnp.pad(Wf1[8:16], ((0, 8), (0, 0)))
    wea = Wf1[16:22]
    out = _edge_mlp(g, ea6, wsrc, wdst, wea, bf1.reshape(1, -1),
                    Wf2.reshape(1, -1), bf2.reshape(1, 1), e)
    return out[:, 0]


# speedup vs baseline: 4.5808x; 1.5660x over previous
import functools

import jax
import jax.numpy as jnp
from jax import lax
from jax.experimental import pallas as pl
from jax.experimental.pallas import tpu as pltpu
from jax.experimental.pallas import tpu_sc as plsc

_K = 16
_BIGF = 1e30
_IBIG = 0x3FFFFFFF


def _bdot(a, b):
    return lax.dot_general(
        a.astype(jnp.bfloat16), b.astype(jnp.bfloat16),
        (((1,), (0,)), ((), ())),
        preferred_element_type=jnp.float32)


_G = 128


def _dist_groups_body(xr_ref, xf_ref, dist_ref, chunk_ref,
                      *, n, r, c, nc, ngrp):
    xr = xr_ref[...]
    xrb = xr.astype(jnp.bfloat16)
    sqr = jnp.sum(xr * xr, axis=1, keepdims=True)
    col_iota = lax.broadcasted_iota(jnp.int32, (r, c), 1)

    gm_parts = []
    for cb in range(nc):
        xc = xf_ref[cb * c:(cb + 1) * c, :]
        sqc = jnp.sum(xc * xc, axis=1)
        dot = lax.dot_general(xrb, xc.astype(jnp.bfloat16),
                              (((1,), (1,)), ((), ())),
                              preferred_element_type=jnp.float32)
        dist = (sqr + sqc[None, :]) - 2.0 * dot
        dist = jnp.where(col_iota + cb * c < n, dist, _BIGF)
        dist_ref[:, cb * c:(cb + 1) * c] = dist
        gm_parts.append(jnp.min(dist.reshape(r, c // _G, _G), axis=2))
    gm = jnp.concatenate(gm_parts, axis=1)

    gi = lax.broadcasted_iota(jnp.int32, (r, ngrp), 1)
    gs = []
    for _s in range(_K):
        m = jnp.min(gm, axis=1, keepdims=True)
        g = jnp.min(jnp.where(gm == m, gi, _IBIG), axis=1, keepdims=True)
        gs.append(g)
        gm = jnp.where(gi == g, _BIGF, gm)
    gsel = jnp.concatenate(gs, axis=1)
    crow = (pl.program_id(0) * r
            + lax.broadcasted_iota(jnp.int32, (r, _K), 0))
    chunk_ref[...] = crow * ngrp + gsel


def _topk_final_body(cand_ref, chunk_ref, idx_ref, *, rb, ngrp):
    cand = cand_ref[...]
    w = _K * _G
    lane = lax.broadcasted_iota(jnp.int32, (rb, w), 1)
    grp = chunk_ref[...] % ngrp
    gcol = (jnp.broadcast_to(grp[:, :, None], (rb, _K, _G)).reshape(rb, w) * _G
            + lane % _G)
    cv = cand
    js = []
    for _s in range(_K):
        m = jnp.min(cv, axis=1, keepdims=True)
        j = jnp.min(jnp.where(cv == m, gcol, _IBIG), axis=1, keepdims=True)
        js.append(j)
        cv = jnp.where(gcol == j, _BIGF, cv)
    idx_ref[...] = jnp.concatenate(js, axis=1)


def _topk(xpad, n, r=128, c=2048):
    npad, d = xpad.shape
    c = min(c, npad)
    nc = npad // c
    ngrp = npad // _G
    body = functools.partial(_dist_groups_body, n=n, r=r, c=c, nc=nc,
                             ngrp=ngrp)
    dist, chunkidx = pl.pallas_call(
        body,
        grid=(npad // r,),
        in_specs=[
            pl.BlockSpec((r, d), lambda i: (i, 0)),
            pl.BlockSpec((npad, d), lambda i: (0, 0)),
        ],
        out_specs=[
            pl.BlockSpec((r, npad), lambda i: (i, 0)),
            pl.BlockSpec((r, _K), lambda i: (i, 0)),
        ],
        out_shape=[
            jax.ShapeDtypeStruct((npad, npad), jnp.float32),
            jax.ShapeDtypeStruct((npad, _K), jnp.int32),
        ],
    )(xpad, xpad)

    cidx = chunkidx[:n].reshape(-1)
    cand = _gather_rows(dist.reshape(npad * ngrp, _G), cidx)
    cand = cand.reshape(n, _K * _G)

    rb = _block_div(n, 512)
    fbody = functools.partial(_topk_final_body, rb=rb, ngrp=ngrp)
    return pl.pallas_call(
        fbody,
        grid=(n // rb,),
        in_specs=[
            pl.BlockSpec((rb, _K * _G), lambda i: (i, 0)),
            pl.BlockSpec((rb, _K), lambda i: (i, 0)),
        ],
        out_specs=pl.BlockSpec((rb, _K), lambda i: (i, 0)),
        out_shape=jax.ShapeDtypeStruct((n, _K), jnp.int32),
    )(cand, chunkidx[:n])


def _pick_chunk(bw, dw):
    best = 8
    for cand in range(8, bw + 1, 8):
        if bw % cand == 0 and cand * (dw + 1) * 4 <= 516000:
            best = cand
    return best


def _gather_rows(table, idx):
    v, dw = table.shape
    b = idx.shape[0]
    info = plsc.get_sparse_core_info()
    nw = info.num_cores * info.num_subcores
    bw = b // nw
    chunk = _pick_chunk(bw, dw)
    nch = bw // chunk
    mesh = plsc.VectorSubcoreMesh(core_axis_name="c", subcore_axis_name="s")

    @functools.partial(
        pl.kernel, mesh=mesh,
        out_type=jax.ShapeDtypeStruct((b, dw), jnp.float32),
        compiler_params=pltpu.CompilerParams(use_tc_tiling_on_sc=False),
        scratch_types=[
            pltpu.VMEM((chunk,), jnp.int32),
            pltpu.VMEM((chunk, dw), jnp.float32),
            pltpu.SemaphoreType.DMA,
        ],
    )
    def gk(table_hbm, idx_hbm, out_hbm, idx_v, rows_v, sem):
        wid = lax.axis_index("s") * info.num_cores + lax.axis_index("c")
        base = wid * bw
        for ch in range(nch):
            off = base + ch * chunk
            pltpu.sync_copy(idx_hbm.at[pl.ds(off, chunk)], idx_v)
            pltpu.async_copy(table_hbm.at[idx_v], rows_v, sem).wait()
            pltpu.sync_copy(rows_v, out_hbm.at[pl.ds(off, chunk)])

    return gk(table, idx)


def _msg_max_body(x_ref, xg_ref, w1t_ref, w1b_ref, b1_ref, w2_ref, b2_ref,
                  o_ref, *, pad_to):
    xi = x_ref[...]
    w1b = w1b_ref[...].astype(jnp.bfloat16)
    w2 = w2_ref[...].astype(jnp.bfloat16)
    a = _bdot(xi, w1t_ref[...]) + b1_ref[...]
    acc = None
    for k in range(_K):
        diff = xg_ref[:, k, :] - xi
        z = lax.dot_general(diff.astype(jnp.bfloat16), w1b,
                            (((1,), (0,)), ((), ())),
                            preferred_element_type=jnp.float32) + a
        z = jax.nn.relu(z)
        z2 = lax.dot_general(z.astype(jnp.bfloat16), w2,
                             (((1,), (0,)), ((), ())),
                             preferred_element_type=jnp.float32)
        acc = z2 if acc is None else jnp.maximum(acc, z2)
    out = jax.nn.relu(acc + b2_ref[...])
    if pad_to is not None:
        rb = out.shape[0]
        out = jnp.concatenate(
            [out, jnp.zeros((rb, pad_to - out.shape[1]), jnp.float32)], axis=1)
    o_ref[...] = out


def _block_div(n, cap):
    best = 8
    for cand in range(8, cap + 1, 8):
        if n % cand == 0:
            best = cand
    return best


def _msg_max(xpad, xg, w1t, w1b, b1, w2, b2, pad_to=None):
    npad, d = xpad.shape
    n = xg.shape[0]
    h = w1t.shape[1]
    out_w = w2.shape[1] if pad_to is None else pad_to
    rb = _block_div(n, 512)
    body = functools.partial(_msg_max_body, pad_to=pad_to)
    return pl.pallas_call(
        body,
        grid=(n // rb,),
        in_specs=[
            pl.BlockSpec((rb, d), lambda i: (i, 0)),
            pl.BlockSpec((rb, _K, d), lambda i: (i, 0, 0)),
            pl.BlockSpec((d, h), lambda i: (0, 0)),
            pl.BlockSpec((d, h), lambda i: (0, 0)),
            pl.BlockSpec((1, h), lambda i: (0, 0)),
            pl.BlockSpec((h, w2.shape[1]), lambda i: (0, 0)),
            pl.BlockSpec((1, w2.shape[1]), lambda i: (0, 0)),
        ],
        out_specs=pl.BlockSpec((rb, out_w), lambda i: (i, 0)),
        out_shape=jax.ShapeDtypeStruct((npad, out_w), jnp.float32),
    )(xpad, xg, w1t, w1b, b1, w2, b2)


def _edge_mlp_body(src_ref, dst_ref, ea_ref, wsrc_ref, wdst_ref, wea_ref,
                   b1_ref, w2_ref, b2_ref, o_ref):
    z = (_bdot(src_ref[...], wsrc_ref[...])
         + _bdot(dst_ref[...], wdst_ref[...])
         + _bdot(ea_ref[...], wea_ref[...])
         + b1_ref[...])
    z = jax.nn.relu(z)
    zb = z.astype(jnp.bfloat16).astype(jnp.float32)
    wb = w2_ref[...].astype(jnp.bfloat16).astype(jnp.float32)
    o_ref[...] = jnp.sum(zb * wb, axis=1, keepdims=True) + b2_ref[...]


def _edge_mlp(g, ea6, wsrc, wdst, wea, b1, w2row, b2, e):
    eb = _block_div(e, 4096)
    nblk = e // eb
    return pl.pallas_call(
        _edge_mlp_body,
        grid=(nblk,),
        in_specs=[
            pl.BlockSpec((eb, 16), lambda i: (i, 0)),
            pl.BlockSpec((eb, 16), lambda i, nblk=nblk: (i + nblk, 0)),
            pl.BlockSpec((eb, 6), lambda i: (i, 0)),
            pl.BlockSpec((16, 16), lambda i: (0, 0)),
            pl.BlockSpec((16, 16), lambda i: (0, 0)),
            pl.BlockSpec((6, 16), lambda i: (0, 0)),
            pl.BlockSpec((1, 16), lambda i: (0, 0)),
            pl.BlockSpec((1, 16), lambda i: (0, 0)),
            pl.BlockSpec((1, 1), lambda i: (0, 0)),
        ],
        out_specs=pl.BlockSpec((eb, 1), lambda i: (i, 0)),
        out_shape=jax.ShapeDtypeStruct((e, 1), jnp.float32),
    )(g, g, ea6, wsrc, wdst, wea, b1, w2row, b2)


def _edge_conv(xpad, w1, b1, w2, b2, n, pad_to=None):
    npad, d = xpad.shape
    idx = _topk(xpad, n)
    idxf = idx[:n].reshape(-1)
    xg = _gather_rows(xpad, idxf).reshape(n, _K, d)
    return _msg_max(xpad, xg, w1[:d], w1[d:], b1.reshape(1, -1),
                    w2, b2.reshape(1, -1), pad_to=pad_to)


def kernel(x, edge_index, edge_attr, year, quarter,
           W1a, b1a, W2a, b2a,
           W1b, b1b, W2b, b2b,
           W1c, b1c, W2c, b2c,
           Wf1, bf1, Wf2, bf2):
    n, d = x.shape
    e = edge_index.shape[1]
    npad = ((n + 511) // 512) * 512

    xpad = jnp.pad(x, ((0, npad - n), (0, 0)))
    h1 = _edge_conv(xpad, W1a, b1a, W2a, b2a, n)
    h2 = _edge_conv(h1, W1b, b1b, W2b, b2b, n)
    h3 = _edge_conv(h2, W1c, b1c, W2c, b2c, n, pad_to=16)

    g = _gather_rows(h3, edge_index.reshape(-1))
    ea6 = jnp.concatenate([edge_attr, year, quarter], axis=1)
    wsrc = jnp.pad(Wf1[:8], ((0, 8), (0, 0)))
    wdst = jnp.pad(Wf1[8:16], ((0, 8), (0, 0)))
    wea = Wf1[16:22]
    out = _edge_mlp(g, ea6, wsrc, wdst, wea, bf1.reshape(1, -1),
                    Wf2.reshape(1, -1), bf2.reshape(1, 1), e)
    return out[:, 0]


# speedup vs baseline: 5.0368x; 1.0996x over previous
import functools

import jax
import jax.numpy as jnp
from jax import lax
from jax.experimental import pallas as pl
from jax.experimental.pallas import tpu as pltpu
from jax.experimental.pallas import tpu_sc as plsc

_K = 16
_BIGF = 1e30
_IBIG = 0x3FFFFFFF


def _bdot(a, b):
    return lax.dot_general(
        a.astype(jnp.bfloat16), b.astype(jnp.bfloat16),
        (((1,), (0,)), ((), ())),
        preferred_element_type=jnp.float32)


_G = 128


def _dist_groups_body(xr_ref, xf_ref, dist_ref, chunk_ref,
                      *, n, r, c, nc, ngrp):
    xr = xr_ref[...]
    xrb = xr.astype(jnp.bfloat16)
    sqr = jnp.sum(xr * xr, axis=1, keepdims=True)
    col_iota = lax.broadcasted_iota(jnp.int32, (r, c), 1)

    gm_parts = []
    for cb in range(nc):
        xc = xf_ref[cb * c:(cb + 1) * c, :]
        sqc = jnp.sum(xc * xc, axis=1)
        dot = lax.dot_general(xrb, xc.astype(jnp.bfloat16),
                              (((1,), (1,)), ((), ())),
                              preferred_element_type=jnp.float32)
        dist = (sqr + sqc[None, :]) - 2.0 * dot
        dist = jnp.where(col_iota + cb * c < n, dist, _BIGF)
        dist_ref[:, cb * c:(cb + 1) * c] = dist
        gm_parts.append(jnp.min(dist.reshape(r, c // _G, _G), axis=2))
    gm = jnp.concatenate(gm_parts, axis=1)

    gi = lax.broadcasted_iota(jnp.int32, (r, ngrp), 1)
    gs = []
    for _s in range(_K):
        m = jnp.min(gm, axis=1, keepdims=True)
        g = jnp.min(jnp.where(gm == m, gi, _IBIG), axis=1, keepdims=True)
        gs.append(g)
        gm = jnp.where(gi == g, _BIGF, gm)
    gsel = jnp.concatenate(gs, axis=1)
    crow = (pl.program_id(0) * r
            + lax.broadcasted_iota(jnp.int32, (r, _K), 0))
    chunk_ref[...] = crow * ngrp + gsel


def _topk_final_body(cand_ref, chunk_ref, idx_ref, *, rb, ngrp):
    cand = cand_ref[...]
    w = _K * _G
    lane = lax.broadcasted_iota(jnp.int32, (rb, w), 1)
    grp = chunk_ref[...] % ngrp
    gcol = (jnp.broadcast_to(grp[:, :, None], (rb, _K, _G)).reshape(rb, w) * _G
            + lane % _G)
    cv = cand
    js = []
    for _s in range(_K):
        m = jnp.min(cv, axis=1, keepdims=True)
        j = jnp.min(jnp.where(cv == m, gcol, _IBIG), axis=1, keepdims=True)
        js.append(j)
        cv = jnp.where(gcol == j, _BIGF, cv)
    idx_ref[...] = jnp.concatenate(js, axis=1)


def _topk(xpad, n, r=256, c=2048):
    npad, d = xpad.shape
    c = min(c, npad)
    nc = npad // c
    ngrp = npad // _G
    body = functools.partial(_dist_groups_body, n=n, r=r, c=c, nc=nc,
                             ngrp=ngrp)
    dist, chunkidx = pl.pallas_call(
        body,
        grid=(npad // r,),
        in_specs=[
            pl.BlockSpec((r, d), lambda i: (i, 0)),
            pl.BlockSpec((npad, d), lambda i: (0, 0)),
        ],
        out_specs=[
            pl.BlockSpec((r, npad), lambda i: (i, 0)),
            pl.BlockSpec((r, _K), lambda i: (i, 0)),
        ],
        out_shape=[
            jax.ShapeDtypeStruct((npad, npad), jnp.float32),
            jax.ShapeDtypeStruct((npad, _K), jnp.int32),
        ],
    )(xpad, xpad)

    cidx = chunkidx[:n].reshape(-1)
    cand = _gather_rows(dist.reshape(npad * ngrp, _G), cidx)
    cand = cand.reshape(n, _K * _G)

    rb = _block_div(n, 512)
    fbody = functools.partial(_topk_final_body, rb=rb, ngrp=ngrp)
    return pl.pallas_call(
        fbody,
        grid=(n // rb,),
        in_specs=[
            pl.BlockSpec((rb, _K * _G), lambda i: (i, 0)),
            pl.BlockSpec((rb, _K), lambda i: (i, 0)),
        ],
        out_specs=pl.BlockSpec((rb, _K), lambda i: (i, 0)),
        out_shape=jax.ShapeDtypeStruct((n, _K), jnp.int32),
    )(cand, chunkidx[:n])


def _pick_chunk(bw, dw):
    best = 8
    for cand in range(8, bw + 1, 8):
        if bw % cand == 0 and cand * (dw + 1) * 4 <= 516000:
            best = cand
    return best


def _gather_rows(table, idx):
    v, dw = table.shape
    b = idx.shape[0]
    info = plsc.get_sparse_core_info()
    nw = info.num_cores * info.num_subcores
    bw = b // nw
    chunk = _pick_chunk(bw, dw)
    nch = bw // chunk
    mesh = plsc.VectorSubcoreMesh(core_axis_name="c", subcore_axis_name="s")

    @functools.partial(
        pl.kernel, mesh=mesh,
        out_type=jax.ShapeDtypeStruct((b, dw), jnp.float32),
        compiler_params=pltpu.CompilerParams(use_tc_tiling_on_sc=False),
        scratch_types=[
            pltpu.VMEM((chunk,), jnp.int32),
            pltpu.VMEM((chunk, dw), jnp.float32),
            pltpu.SemaphoreType.DMA,
        ],
    )
    def gk(table_hbm, idx_hbm, out_hbm, idx_v, rows_v, sem):
        wid = lax.axis_index("s") * info.num_cores + lax.axis_index("c")
        base = wid * bw
        for ch in range(nch):
            off = base + ch * chunk
            pltpu.sync_copy(idx_hbm.at[pl.ds(off, chunk)], idx_v)
            pltpu.async_copy(table_hbm.at[idx_v], rows_v, sem).wait()
            pltpu.sync_copy(rows_v, out_hbm.at[pl.ds(off, chunk)])

    return gk(table, idx)


def _msg_max_body(x_ref, xg_ref, w1t_ref, w1b_ref, b1_ref, w2_ref, b2_ref,
                  o_ref, *, pad_to):
    xi = x_ref[...]
    w1b = w1b_ref[...].astype(jnp.bfloat16)
    w2 = w2_ref[...].astype(jnp.bfloat16)
    a = _bdot(xi, w1t_ref[...]) + b1_ref[...]
    acc = None
    for k in range(_K):
        diff = xg_ref[:, k, :] - xi
        z = lax.dot_general(diff.astype(jnp.bfloat16), w1b,
                            (((1,), (0,)), ((), ())),
                            preferred_element_type=jnp.float32) + a
        z = jax.nn.relu(z)
        z2 = lax.dot_general(z.astype(jnp.bfloat16), w2,
                             (((1,), (0,)), ((), ())),
                             preferred_element_type=jnp.float32)
        acc = z2 if acc is None else jnp.maximum(acc, z2)
    out = jax.nn.relu(acc + b2_ref[...])
    if pad_to is not None:
        rb = out.shape[0]
        out = jnp.concatenate(
            [out, jnp.zeros((rb, pad_to - out.shape[1]), jnp.float32)], axis=1)
    o_ref[...] = out


def _block_div(n, cap):
    best = 8
    for cand in range(8, cap + 1, 8):
        if n % cand == 0:
            best = cand
    return best


def _msg_max(xpad, xg, w1t, w1b, b1, w2, b2, pad_to=None):
    npad, d = xpad.shape
    n = xg.shape[0]
    h = w1t.shape[1]
    out_w = w2.shape[1] if pad_to is None else pad_to
    rb = _block_div(n, 512)
    body = functools.partial(_msg_max_body, pad_to=pad_to)
    return pl.pallas_call(
        body,
        grid=(n // rb,),
        in_specs=[
            pl.BlockSpec((rb, d), lambda i: (i, 0)),
            pl.BlockSpec((rb, _K, d), lambda i: (i, 0, 0)),
            pl.BlockSpec((d, h), lambda i: (0, 0)),
            pl.BlockSpec((d, h), lambda i: (0, 0)),
            pl.BlockSpec((1, h), lambda i: (0, 0)),
            pl.BlockSpec((h, w2.shape[1]), lambda i: (0, 0)),
            pl.BlockSpec((1, w2.shape[1]), lambda i: (0, 0)),
        ],
        out_specs=pl.BlockSpec((rb, out_w), lambda i: (i, 0)),
        out_shape=jax.ShapeDtypeStruct((npad, out_w), jnp.float32),
    )(xpad, xg, w1t, w1b, b1, w2, b2)


def _edge_mlp_body(src_ref, dst_ref, ea_ref, wsrc_ref, wdst_ref, wea_ref,
                   b1_ref, w2_ref, b2_ref, o_ref):
    z = (_bdot(src_ref[...], wsrc_ref[...])
         + _bdot(dst_ref[...], wdst_ref[...])
         + _bdot(ea_ref[...], wea_ref[...])
         + b1_ref[...])
    z = jax.nn.relu(z)
    zb = z.astype(jnp.bfloat16).astype(jnp.float32)
    wb = w2_ref[...].astype(jnp.bfloat16).astype(jnp.float32)
    o_ref[...] = jnp.sum(zb * wb, axis=1, keepdims=True) + b2_ref[...]


def _edge_mlp(g, ea6, wsrc, wdst, wea, b1, w2row, b2, e):
    eb = _block_div(e, 4096)
    nblk = e // eb
    return pl.pallas_call(
        _edge_mlp_body,
        grid=(nblk,),
        in_specs=[
            pl.BlockSpec((eb, 16), lambda i: (i, 0)),
            pl.BlockSpec((eb, 16), lambda i, nblk=nblk: (i + nblk, 0)),
            pl.BlockSpec((eb, 6), lambda i: (i, 0)),
            pl.BlockSpec((16, 16), lambda i: (0, 0)),
            pl.BlockSpec((16, 16), lambda i: (0, 0)),
            pl.BlockSpec((6, 16), lambda i: (0, 0)),
            pl.BlockSpec((1, 16), lambda i: (0, 0)),
            pl.BlockSpec((1, 16), lambda i: (0, 0)),
            pl.BlockSpec((1, 1), lambda i: (0, 0)),
        ],
        out_specs=pl.BlockSpec((eb, 1), lambda i: (i, 0)),
        out_shape=jax.ShapeDtypeStruct((e, 1), jnp.float32),
    )(g, g, ea6, wsrc, wdst, wea, b1, w2row, b2)


def _edge_conv(xpad, w1, b1, w2, b2, n, pad_to=None):
    npad, d = xpad.shape
    idx = _topk(xpad, n)
    idxf = idx[:n].reshape(-1)
    xg = _gather_rows(xpad, idxf).reshape(n, _K, d)
    return _msg_max(xpad, xg, w1[:d], w1[d:], b1.reshape(1, -1),
                    w2, b2.reshape(1, -1), pad_to=pad_to)


def kernel(x, edge_index, edge_attr, year, quarter,
           W1a, b1a, W2a, b2a,
           W1b, b1b, W2b, b2b,
           W1c, b1c, W2c, b2c,
           Wf1, bf1, Wf2, bf2):
    n, d = x.shape
    e = edge_index.shape[1]
    npad = ((n + 511) // 512) * 512

    xpad = jnp.pad(x, ((0, npad - n), (0, 0)))
    h1 = _edge_conv(xpad, W1a, b1a, W2a, b2a, n)
    h2 = _edge_conv(h1, W1b, b1b, W2b, b2b, n)
    h3 = _edge_conv(h2, W1c, b1c, W2c, b2c, n, pad_to=16)

    g = _gather_rows(h3, edge_index.reshape(-1))
    ea6 = jnp.concatenate([edge_attr, year, quarter], axis=1)
    wsrc = jnp.pad(Wf1[:8], ((0, 8), (0, 0)))
    wdst = jnp.pad(Wf1[8:16], ((0, 8), (0, 0)))
    wea = Wf1[16:22]
    out = _edge_mlp(g, ea6, wsrc, wdst, wea, bf1.reshape(1, -1),
                    Wf2.reshape(1, -1), bf2.reshape(1, 1), e)
    return out[:, 0]


# speedup vs baseline: 5.2720x; 1.0467x over previous
import functools

import jax
import jax.numpy as jnp
from jax import lax
from jax.experimental import pallas as pl
from jax.experimental.pallas import tpu as pltpu
from jax.experimental.pallas import tpu_sc as plsc

_K = 16
_BIGF = 1e30
_IBIG = 0x3FFFFFFF


def _bdot(a, b):
    return lax.dot_general(
        a.astype(jnp.bfloat16), b.astype(jnp.bfloat16),
        (((1,), (0,)), ((), ())),
        preferred_element_type=jnp.float32)


_G = 128


def _dist_groups_body(xr_ref, xf_ref, dist_ref, chunk_ref,
                      *, n, r, c, nc, ngrp):
    xr = xr_ref[...]
    xrb = xr.astype(jnp.bfloat16)
    sqr = jnp.sum(xr * xr, axis=1, keepdims=True)
    col_iota = lax.broadcasted_iota(jnp.int32, (r, c), 1)

    gm_parts = []
    for cb in range(nc):
        xc = xf_ref[cb * c:(cb + 1) * c, :]
        sqc = jnp.sum(xc * xc, axis=1)
        dot = lax.dot_general(xrb, xc.astype(jnp.bfloat16),
                              (((1,), (1,)), ((), ())),
                              preferred_element_type=jnp.float32)
        dist = (sqr + sqc[None, :]) - 2.0 * dot
        dist = jnp.where(col_iota + cb * c < n, dist, _BIGF)
        dist_ref[:, cb * c:(cb + 1) * c] = dist
        gm_parts.append(jnp.min(dist.reshape(r, c // _G, _G), axis=2))
    gm = jnp.concatenate(gm_parts, axis=1)

    gi = lax.broadcasted_iota(jnp.int32, (r, ngrp), 1)
    gs = []
    for _s in range(_K):
        m = jnp.min(gm, axis=1, keepdims=True)
        g = jnp.min(jnp.where(gm == m, gi, _IBIG), axis=1, keepdims=True)
        gs.append(g)
        gm = jnp.where(gi == g, _BIGF, gm)
    gsel = jnp.concatenate(gs, axis=1)
    crow = (pl.program_id(0) * r
            + lax.broadcasted_iota(jnp.int32, (r, _K), 0))
    chunk_ref[...] = crow * ngrp + gsel


def _topk_final_body(cand_ref, chunk_ref, idx_ref, *, rb, ngrp):
    cand = cand_ref[...]
    w = _K * _G
    lane = lax.broadcasted_iota(jnp.int32, (rb, w), 1)
    grp = chunk_ref[...] % ngrp
    gcol = (jnp.broadcast_to(grp[:, :, None], (rb, _K, _G)).reshape(rb, w) * _G
            + lane % _G)
    cv = cand
    js = []
    for _s in range(_K):
        m = jnp.min(cv, axis=1, keepdims=True)
        j = jnp.min(jnp.where(cv == m, gcol, _IBIG), axis=1, keepdims=True)
        js.append(j)
        cv = jnp.where(gcol == j, _BIGF, cv)
    idx_ref[...] = jnp.concatenate(js, axis=1)


def _topk(xpad, n, r=512, c=2048):
    npad, d = xpad.shape
    c = min(c, npad)
    nc = npad // c
    ngrp = npad // _G
    body = functools.partial(_dist_groups_body, n=n, r=r, c=c, nc=nc,
                             ngrp=ngrp)
    dist, chunkidx = pl.pallas_call(
        body,
        grid=(npad // r,),
        in_specs=[
            pl.BlockSpec((r, d), lambda i: (i, 0)),
            pl.BlockSpec((npad, d), lambda i: (0, 0)),
        ],
        out_specs=[
            pl.BlockSpec((r, npad), lambda i: (i, 0)),
            pl.BlockSpec((r, _K), lambda i: (i, 0)),
        ],
        out_shape=[
            jax.ShapeDtypeStruct((npad, npad), jnp.float32),
            jax.ShapeDtypeStruct((npad, _K), jnp.int32),
        ],
    )(xpad, xpad)

    cidx = chunkidx[:n].reshape(-1)
    cand = _gather_rows(dist.reshape(npad * ngrp, _G), cidx)
    cand = cand.reshape(n, _K * _G)

    rb = _block_div(n, 512)
    fbody = functools.partial(_topk_final_body, rb=rb, ngrp=ngrp)
    return pl.pallas_call(
        fbody,
        grid=(n // rb,),
        in_specs=[
            pl.BlockSpec((rb, _K * _G), lambda i: (i, 0)),
            pl.BlockSpec((rb, _K), lambda i: (i, 0)),
        ],
        out_specs=pl.BlockSpec((rb, _K), lambda i: (i, 0)),
        out_shape=jax.ShapeDtypeStruct((n, _K), jnp.int32),
    )(cand, chunkidx[:n])


def _pick_chunk(bw, dw):
    best = 8
    for cand in range(8, bw + 1, 8):
        if bw % cand == 0 and cand * (dw + 1) * 4 <= 516000:
            best = cand
    return best


def _gather_rows(table, idx):
    v, dw = table.shape
    b = idx.shape[0]
    info = plsc.get_sparse_core_info()
    nw = info.num_cores * info.num_subcores
    bw = b // nw
    chunk = _pick_chunk(bw, dw)
    nch = bw // chunk
    mesh = plsc.VectorSubcoreMesh(core_axis_name="c", subcore_axis_name="s")

    @functools.partial(
        pl.kernel, mesh=mesh,
        out_type=jax.ShapeDtypeStruct((b, dw), jnp.float32),
        compiler_params=pltpu.CompilerParams(use_tc_tiling_on_sc=False),
        scratch_types=[
            pltpu.VMEM((chunk,), jnp.int32),
            pltpu.VMEM((chunk, dw), jnp.float32),
            pltpu.SemaphoreType.DMA,
        ],
    )
    def gk(table_hbm, idx_hbm, out_hbm, idx_v, rows_v, sem):
        wid = lax.axis_index("s") * info.num_cores + lax.axis_index("c")
        base = wid * bw
        for ch in range(nch):
            off = base + ch * chunk
            pltpu.sync_copy(idx_hbm.at[pl.ds(off, chunk)], idx_v)
            pltpu.async_copy(table_hbm.at[idx_v], rows_v, sem).wait()
            pltpu.sync_copy(rows_v, out_hbm.at[pl.ds(off, chunk)])

    return gk(table, idx)


def _msg_max_body(x_ref, xg_ref, w1t_ref, w1b_ref, b1_ref, w2_ref, b2_ref,
                  o_ref, *, pad_to):
    xi = x_ref[...]
    w1b = w1b_ref[...].astype(jnp.bfloat16)
    w2 = w2_ref[...].astype(jnp.bfloat16)
    a = _bdot(xi, w1t_ref[...]) + b1_ref[...]
    acc = None
    for k in range(_K):
        diff = xg_ref[:, k, :] - xi
        z = lax.dot_general(diff.astype(jnp.bfloat16), w1b,
                            (((1,), (0,)), ((), ())),
                            preferred_element_type=jnp.float32) + a
        z = jax.nn.relu(z)
        z2 = lax.dot_general(z.astype(jnp.bfloat16), w2,
                             (((1,), (0,)), ((), ())),
                             preferred_element_type=jnp.float32)
        acc = z2 if acc is None else jnp.maximum(acc, z2)
    out = jax.nn.relu(acc + b2_ref[...])
    if pad_to is not None:
        rb = out.shape[0]
        out = jnp.concatenate(
            [out, jnp.zeros((rb, pad_to - out.shape[1]), jnp.float32)], axis=1)
    o_ref[...] = out


def _block_div(n, cap):
    best = 8
    for cand in range(8, cap + 1, 8):
        if n % cand == 0:
            best = cand
    return best


def _msg_max(xpad, xg, w1t, w1b, b1, w2, b2, pad_to=None):
    npad, d = xpad.shape
    n = xg.shape[0]
    h = w1t.shape[1]
    out_w = w2.shape[1] if pad_to is None else pad_to
    rb = _block_div(n, 512)
    body = functools.partial(_msg_max_body, pad_to=pad_to)
    return pl.pallas_call(
        body,
        grid=(n // rb,),
        in_specs=[
            pl.BlockSpec((rb, d), lambda i: (i, 0)),
            pl.BlockSpec((rb, _K, d), lambda i: (i, 0, 0)),
            pl.BlockSpec((d, h), lambda i: (0, 0)),
            pl.BlockSpec((d, h), lambda i: (0, 0)),
            pl.BlockSpec((1, h), lambda i: (0, 0)),
            pl.BlockSpec((h, w2.shape[1]), lambda i: (0, 0)),
            pl.BlockSpec((1, w2.shape[1]), lambda i: (0, 0)),
        ],
        out_specs=pl.BlockSpec((rb, out_w), lambda i: (i, 0)),
        out_shape=jax.ShapeDtypeStruct((npad, out_w), jnp.float32),
    )(xpad, xg, w1t, w1b, b1, w2, b2)


def _edge_mlp_body(src_ref, dst_ref, ea_ref, wsrc_ref, wdst_ref, wea_ref,
                   b1_ref, w2_ref, b2_ref, o_ref):
    z = (_bdot(src_ref[...], wsrc_ref[...])
         + _bdot(dst_ref[...], wdst_ref[...])
         + _bdot(ea_ref[...], wea_ref[...])
         + b1_ref[...])
    z = jax.nn.relu(z)
    zb = z.astype(jnp.bfloat16).astype(jnp.float32)
    wb = w2_ref[...].astype(jnp.bfloat16).astype(jnp.float32)
    o_ref[...] = jnp.sum(zb * wb, axis=1, keepdims=True) + b2_ref[...]


def _edge_mlp(g, ea6, wsrc, wdst, wea, b1, w2row, b2, e):
    eb = _block_div(e, 4096)
    nblk = e // eb
    return pl.pallas_call(
        _edge_mlp_body,
        grid=(nblk,),
        in_specs=[
            pl.BlockSpec((eb, 16), lambda i: (i, 0)),
            pl.BlockSpec((eb, 16), lambda i, nblk=nblk: (i + nblk, 0)),
            pl.BlockSpec((eb, 6), lambda i: (i, 0)),
            pl.BlockSpec((16, 16), lambda i: (0, 0)),
            pl.BlockSpec((16, 16), lambda i: (0, 0)),
            pl.BlockSpec((6, 16), lambda i: (0, 0)),
            pl.BlockSpec((1, 16), lambda i: (0, 0)),
            pl.BlockSpec((1, 16), lambda i: (0, 0)),
            pl.BlockSpec((1, 1), lambda i: (0, 0)),
        ],
        out_specs=pl.BlockSpec((eb, 1), lambda i: (i, 0)),
        out_shape=jax.ShapeDtypeStruct((e, 1), jnp.float32),
    )(g, g, ea6, wsrc, wdst, wea, b1, w2row, b2)


def _edge_conv(xpad, w1, b1, w2, b2, n, pad_to=None):
    npad, d = xpad.shape
    idx = _topk(xpad, n)
    idxf = idx[:n].reshape(-1)
    xg = _gather_rows(xpad, idxf).reshape(n, _K, d)
    return _msg_max(xpad, xg, w1[:d], w1[d:], b1.reshape(1, -1),
                    w2, b2.reshape(1, -1), pad_to=pad_to)


def kernel(x, edge_index, edge_attr, year, quarter,
           W1a, b1a, W2a, b2a,
           W1b, b1b, W2b, b2b,
           W1c, b1c, W2c, b2c,
           Wf1, bf1, Wf2, bf2):
    n, d = x.shape
    e = edge_index.shape[1]
    npad = ((n + 511) // 512) * 512

    xpad = jnp.pad(x, ((0, npad - n), (0, 0)))
    h1 = _edge_conv(xpad, W1a, b1a, W2a, b2a, n)
    h2 = _edge_conv(h1, W1b, b1b, W2b, b2b, n)
    h3 = _edge_conv(h2, W1c, b1c, W2c, b2c, n, pad_to=16)

    g = _gather_rows(h3, edge_index.reshape(-1))
    ea6 = jnp.concatenate([edge_attr, year, quarter], axis=1)
    wsrc = jnp.pad(Wf1[:8], ((0, 8), (0, 0)))
    wdst = jnp.pad(Wf1[8:16], ((0, 8), (0, 0)))
    wea = Wf1[16:22]
    out = _edge_mlp(g, ea6, wsrc, wdst, wea, bf1.reshape(1, -1),
                    Wf2.reshape(1, -1), bf2.reshape(1, 1), e)
    return out[:, 0]
